# Initial kernel scaffold; baseline (speedup 1.0000x reference)
import jax
import jax.numpy as jnp
from jax.experimental import pallas as pl


def kernel(X, edge_index, edge_weight, edge_attr, q, W0, b0, W1, b1, W2, b2, T0, T1, T2, T3, T4, T5):
    raise NotImplementedError("write your pallas kernel here")


# speedup vs baseline: 20.2641x; 20.2641x over previous
import functools

import jax
import jax.numpy as jnp
from jax import lax
from jax.experimental import pallas as pl
from jax.experimental.pallas import tpu as pltpu
from jax.experimental.pallas import tpu_sc as plsc

CUTOFF = 5.0

_GRP = (0, 1, 1, 1, 2, 2, 2, 2, 2)


def _dotT(x, w):
    return lax.dot_general(x, w, (((1,), (1,)), ((), ())),
                           preferred_element_type=jnp.float32)


def _mlp_body(ea_ref, ew_ref, w0_ref, b0_ref, w1_ref, b1_ref, w2_ref, b2_ref,
              *o_ref):
    x = ea_ref[...]
    y = jax.nn.silu(_dotT(x, w0_ref[...]) + b0_ref[...])
    y = jax.nn.silu(_dotT(y, w1_ref[...]) + b1_ref[...])
    y = jax.nn.silu(_dotT(y, w2_ref[...]) + b2_ref[...])
    ew = ew_ref[...]
    c = 0.5 * (jnp.cos(jnp.pi * ew / CUTOFF) + 1.0)
    c = c * (ew < CUTOFF).astype(jnp.float32)
    y = y * c
    H = o_ref[0].shape[-1]
    o_ref[0][...] = y[:, :H]
    o_ref[1][...] = y[:, H:2 * H]
    o_ref[2][...] = y[:, 2 * H:]


def _run_mlp(edge_attr, edge_weight, W0, b0, W1, b1, W2, b2):
    E, R = edge_attr.shape
    H = W0.shape[0]
    BE = 512
    j = jnp.arange(3 * H)
    perm = (j % H) * 3 + j // H
    W2p = W2[perm]
    b2p = b2[perm]
    ew = edge_weight.reshape(E, 1)
    grid = (E // BE,)
    out = pl.pallas_call(
        _mlp_body,
        grid=grid,
        in_specs=[
            pl.BlockSpec((BE, R), lambda i: (i, 0)),
            pl.BlockSpec((BE, 1), lambda i: (i, 0)),
            pl.BlockSpec((H, R), lambda i: (0, 0)),
            pl.BlockSpec((1, H), lambda i: (0, 0)),
            pl.BlockSpec((2 * H, H), lambda i: (0, 0)),
            pl.BlockSpec((1, 2 * H), lambda i: (0, 0)),
            pl.BlockSpec((3 * H, 2 * H), lambda i: (0, 0)),
            pl.BlockSpec((1, 3 * H), lambda i: (0, 0)),
        ],
        out_specs=[pl.BlockSpec((BE, H), lambda i: (i, 0))] * 3,
        out_shape=[jax.ShapeDtypeStruct((E, H), jnp.float32)] * 3,
    )(edge_attr, ew, W0, b0.reshape(1, H), W1, b1.reshape(1, 2 * H),
      W2p, b2p.reshape(1, 3 * H))
    return out


def _prep_body(x_ref, t0_ref, t1_ref, t2_ref, *o_refs):
    x = x_ref[...]
    tn = jnp.sum(x * x, axis=1) + 1.0
    inv = 1.0 / tn
    e = [x[:, k, :] * inv for k in range(9)]
    lam = (e[0] + e[4] + e[8]) * (1.0 / 3.0)
    a01 = 0.5 * (e[1] - e[3])
    a02 = 0.5 * (e[2] - e[6])
    a12 = 0.5 * (e[5] - e[7])
    s00 = e[0] - lam
    s11 = e[4] - lam
    s01 = 0.5 * (e[1] + e[3])
    s02 = 0.5 * (e[2] + e[6])
    s12 = 0.5 * (e[5] + e[7])
    t0 = t0_ref[...]
    t1 = t1_ref[...]
    t2 = t2_ref[...]
    o_refs[0][...] = _dotT(lam, t0)
    o_refs[1][...] = _dotT(a01, t1)
    o_refs[2][...] = _dotT(a02, t1)
    o_refs[3][...] = _dotT(a12, t1)
    o_refs[4][...] = _dotT(s00, t2)
    o_refs[5][...] = _dotT(s11, t2)
    o_refs[6][...] = _dotT(s01, t2)
    o_refs[7][...] = _dotT(s02, t2)
    o_refs[8][...] = _dotT(s12, t2)


def _run_prep(Xt, T0, T1, T2):
    N, _, H = Xt.shape
    BN = 1000
    grid = (N // BN,)
    outs = pl.pallas_call(
        _prep_body,
        grid=grid,
        in_specs=[
            pl.BlockSpec((BN, 9, H), lambda i: (i, 0, 0)),
            pl.BlockSpec((H, H), lambda i: (0, 0)),
            pl.BlockSpec((H, H), lambda i: (0, 0)),
            pl.BlockSpec((H, H), lambda i: (0, 0)),
        ],
        out_specs=[pl.BlockSpec((BN, H), lambda i: (i, 0))] * 9,
        out_shape=[jax.ShapeDtypeStruct((N, H), jnp.float32)] * 9,
    )(Xt, T0, T1, T2)
    return outs


def _make_sc_mp(N, E, H, n_batch, B):
    NT = 32
    ept = E // NT
    rpt = 640
    NA = 16 * rpt
    tail = N - 15 * rpt
    mesh = plsc.VectorSubcoreMesh(core_axis_name="c", subcore_axis_name="s")

    @functools.partial(
        pl.kernel,
        mesh=mesh,
        out_type=[jax.ShapeDtypeStruct((2, N, H), jnp.float32)] * 9,
        scratch_types=[
            pltpu.VMEM((n_batch, B), jnp.int32),
            pltpu.VMEM((n_batch, B), jnp.int32),
            pltpu.VMEM((B, H), jnp.float32),
            pltpu.VMEM((B, H), jnp.float32),
            pltpu.VMEM((64, H), jnp.float32),
            pltpu.VMEM_SHARED((NA, H), jnp.float32),
            pltpu.SemaphoreType.DMA,
        ],
    )
    def sc_mp(src_hbm, dst_hbm, ea0, ea1, ea2, g0, g1, g2, g3, g4, g5, g6,
              g7, g8, o0, o1, o2, o3, o4, o5, o6, o7, o8,
              src_v, dst_v, rows_v, coef_v, zeros_v, acc, sem):
        ea_refs = (ea0, ea1, ea2)
        g_refs = (g0, g1, g2, g3, g4, g5, g6, g7, g8)
        o_refs = (o0, o1, o2, o3, o4, o5, o6, o7, o8)
        cid = lax.axis_index("c")
        sid = lax.axis_index("s")
        wid = cid * 16 + sid

        def zrow(i, _):
            for k in range(H // 16):
                zeros_v[i, pl.ds(k * 16, 16)] = jnp.zeros((16,), jnp.float32)
            return 0
        lax.fori_loop(0, 64, zrow, 0)

        pltpu.sync_copy(src_hbm.at[wid], src_v)
        pltpu.sync_copy(dst_hbm.at[wid], dst_v)

        for comp in range(9):
            ea_hbm = ea_refs[_GRP[comp]]
            for z in range(rpt // 64):
                pltpu.sync_copy(zeros_v,
                                acc.at[pl.ds(sid * rpt + z * 64, 64)])
            plsc.subcore_barrier()

            def batch(j, _):
                eoff = wid * ept + j * B
                pltpu.async_copy(g_refs[comp].at[src_v.at[j]], rows_v,
                                 sem).wait()
                pltpu.sync_copy(ea_hbm.at[pl.ds(eoff, B)], coef_v)

                def edge(e, _):
                    for k in range(H // 16):
                        sl = pl.ds(k * 16, 16)
                        rows_v[e, sl] = rows_v[e, sl] * coef_v[e, sl]
                    return 0
                lax.fori_loop(0, B, edge, 0)
                pltpu.sync_copy(rows_v, acc.at[dst_v.at[j]], add=True)
                return 0
            lax.fori_loop(0, n_batch, batch, 0)
            plsc.subcore_barrier()
            @pl.when(sid < 15)
            def _():
                pltpu.sync_copy(
                    acc.at[pl.ds(sid * rpt, rpt)],
                    o_refs[comp].at[cid, pl.ds(sid * rpt, rpt)])

            @pl.when(sid == 15)
            def _():
                pltpu.sync_copy(
                    acc.at[pl.ds(15 * rpt, tail)],
                    o_refs[comp].at[cid, pl.ds(15 * rpt, tail)])

    return sc_mp


def _final_body(x_ref, q_ref, t3_ref, t4_ref, t5_ref, *refs):
    g = [refs[k][...] for k in range(9)]
    m = [refs[9 + k][0] + refs[9 + k][1] for k in range(9)]
    x = x_ref[...]
    tn = jnp.sum(x * x, axis=1) + 1.0
    inv = 1.0 / tn
    xn = [x[:, k, :] * inv for k in range(9)]

    def full(lam, a01, a02, a12, s00, s11, s01, s02, s12):
        s22 = -(s00 + s11)
        return [[lam + s00, a01 + s01, a02 + s02],
                [s01 - a01, lam + s11, a12 + s12],
                [s02 - a02, s12 - a12, lam + s22]]

    Y = full(*g)
    M = full(*m)
    q = q_ref[...]
    qf = 1.0 + 0.1 * q
    Z = [[None] * 3 for _ in range(3)]
    for i in range(3):
        for jj in range(3):
            acc = None
            for k in range(3):
                term = M[i][k] * Y[k][jj] + Y[i][k] * M[k][jj]
                acc = term if acc is None else acc + term
            Z[i][jj] = qf * acc
    nrm = None
    for i in range(3):
        for jj in range(3):
            t = Z[i][jj] * Z[i][jj]
            nrm = t if nrm is None else nrm + t
    invn = 1.0 / (nrm + 1.0)
    lam = (Z[0][0] + Z[1][1] + Z[2][2]) * (1.0 / 3.0) * invn
    a01 = 0.5 * (Z[0][1] - Z[1][0]) * invn
    a02 = 0.5 * (Z[0][2] - Z[2][0]) * invn
    a12 = 0.5 * (Z[1][2] - Z[2][1]) * invn
    s00 = Z[0][0] * invn - lam
    s11 = Z[1][1] * invn - lam
    s01 = 0.5 * (Z[0][1] + Z[1][0]) * invn
    s02 = 0.5 * (Z[0][2] + Z[2][0]) * invn
    s12 = 0.5 * (Z[1][2] + Z[2][1]) * invn
    t3 = t3_ref[...]
    t4 = t4_ref[...]
    t5 = t5_ref[...]
    D = full(_dotT(lam, t3), _dotT(a01, t4), _dotT(a02, t4), _dotT(a12, t4),
             _dotT(s00, t5), _dotT(s11, t5), _dotT(s01, t5), _dotT(s02, t5),
             _dotT(s12, t5))
    o_ref = refs[18]
    for i in range(3):
        for jj in range(3):
            dd = None
            for k in range(3):
                t = D[i][k] * D[k][jj]
                dd = t if dd is None else dd + t
            o_ref[:, 3 * i + jj, :] = xn[3 * i + jj] + D[i][jj] + qf * dd


def _run_final(Xt, q, T3, T4, T5, g_comps, m_parts):
    N, _, H = Xt.shape
    BN = 200
    grid = (N // BN,)
    out = pl.pallas_call(
        _final_body,
        grid=grid,
        in_specs=[
            pl.BlockSpec((BN, 9, H), lambda i: (i, 0, 0)),
            pl.BlockSpec((BN, 1), lambda i: (i, 0)),
            pl.BlockSpec((H, H), lambda i: (0, 0)),
            pl.BlockSpec((H, H), lambda i: (0, 0)),
            pl.BlockSpec((H, H), lambda i: (0, 0)),
        ] + [pl.BlockSpec((BN, H), lambda i: (i, 0))] * 9
          + [pl.BlockSpec((2, BN, H), lambda i: (0, i, 0))] * 9,
        out_specs=pl.BlockSpec((BN, 9, H), lambda i: (i, 0, 0)),
        out_shape=jax.ShapeDtypeStruct((N, 9, H), jnp.float32),
    )(Xt, q.reshape(N, 1), T3, T4, T5, *g_comps, *m_parts)
    return out


def kernel(X, edge_index, edge_weight, edge_attr, q,
           W0, b0, W1, b1, W2, b2, T0, T1, T2, T3, T4, T5):
    N, H = X.shape[0], X.shape[1]
    E = edge_index.shape[1]
    NT = 32
    B = 64
    n_batch = -(-E // (NT * B))
    Ep = NT * B * n_batch
    pad = Ep - E

    ea_in = jnp.pad(edge_attr, ((0, pad), (0, 0)))
    ew_in = jnp.pad(edge_weight, (0, pad), constant_values=2.0 * CUTOFF)
    ea = _run_mlp(ea_in, ew_in, W0, b0, W1, b1, W2, b2)

    Xt = X.reshape(N, H, 9).transpose(0, 2, 1)
    g_comps = _run_prep(Xt, T0, T1, T2)

    ei = jnp.pad(edge_index.astype(jnp.int32), ((0, 0), (0, pad)))
    dst = ei[0].reshape(NT, n_batch, B)
    src = ei[1].reshape(NT, n_batch, B)
    sc_mp = _make_sc_mp(N, Ep, H, n_batch, B)
    m_parts = sc_mp(src, dst, *ea, *g_comps)

    out = _run_final(Xt, q, T3, T4, T5, g_comps, m_parts)
    return out.transpose(0, 2, 1).reshape(N, H, 3, 3)


# speedup vs baseline: 22.8708x; 1.1286x over previous
import functools

import jax
import jax.numpy as jnp
from jax import lax
from jax.experimental import pallas as pl
from jax.experimental.pallas import tpu as pltpu
from jax.experimental.pallas import tpu_sc as plsc

CUTOFF = 5.0

_GRP = (0, 1, 1, 1, 2, 2, 2, 2, 2)


def _dotT(x, w):
    return lax.dot_general(x, w, (((1,), (1,)), ((), ())),
                           preferred_element_type=jnp.float32)


def _mlp_body(ea_ref, ew_ref, w0_ref, b0_ref, w1_ref, b1_ref, w2_ref, b2_ref,
              *o_ref):
    x = ea_ref[...]
    y = jax.nn.silu(_dotT(x, w0_ref[...]) + b0_ref[...])
    y = jax.nn.silu(_dotT(y, w1_ref[...]) + b1_ref[...])
    y = jax.nn.silu(_dotT(y, w2_ref[...]) + b2_ref[...])
    ew = ew_ref[...]
    c = 0.5 * (jnp.cos(jnp.pi * ew / CUTOFF) + 1.0)
    c = c * (ew < CUTOFF).astype(jnp.float32)
    y = y * c
    H = o_ref[0].shape[-1]
    o_ref[0][...] = y[:, :H]
    o_ref[1][...] = y[:, H:2 * H]
    o_ref[2][...] = y[:, 2 * H:]


def _run_mlp(edge_attr, edge_weight, W0, b0, W1, b1, W2, b2):
    E, R = edge_attr.shape
    H = W0.shape[0]
    BE = 512
    j = jnp.arange(3 * H)
    perm = (j % H) * 3 + j // H
    W2p = W2[perm]
    b2p = b2[perm]
    ew = edge_weight.reshape(E, 1)
    grid = (E // BE,)
    out = pl.pallas_call(
        _mlp_body,
        grid=grid,
        in_specs=[
            pl.BlockSpec((BE, R), lambda i: (i, 0)),
            pl.BlockSpec((BE, 1), lambda i: (i, 0)),
            pl.BlockSpec((H, R), lambda i: (0, 0)),
            pl.BlockSpec((1, H), lambda i: (0, 0)),
            pl.BlockSpec((2 * H, H), lambda i: (0, 0)),
            pl.BlockSpec((1, 2 * H), lambda i: (0, 0)),
            pl.BlockSpec((3 * H, 2 * H), lambda i: (0, 0)),
            pl.BlockSpec((1, 3 * H), lambda i: (0, 0)),
        ],
        out_specs=[pl.BlockSpec((BE, H), lambda i: (i, 0))] * 3,
        out_shape=[jax.ShapeDtypeStruct((E, H), jnp.float32)] * 3,
    )(edge_attr, ew, W0, b0.reshape(1, H), W1, b1.reshape(1, 2 * H),
      W2p, b2p.reshape(1, 3 * H))
    return out


def _prep_body(x_ref, t0_ref, t1_ref, t2_ref, *o_refs):
    x = x_ref[...]
    tn = jnp.sum(x * x, axis=1) + 1.0
    inv = 1.0 / tn
    e = [x[:, k, :] * inv for k in range(9)]
    lam = (e[0] + e[4] + e[8]) * (1.0 / 3.0)
    a01 = 0.5 * (e[1] - e[3])
    a02 = 0.5 * (e[2] - e[6])
    a12 = 0.5 * (e[5] - e[7])
    s00 = e[0] - lam
    s11 = e[4] - lam
    s01 = 0.5 * (e[1] + e[3])
    s02 = 0.5 * (e[2] + e[6])
    s12 = 0.5 * (e[5] + e[7])
    t0 = t0_ref[...]
    t1 = t1_ref[...]
    t2 = t2_ref[...]
    o_refs[0][...] = _dotT(lam, t0)
    o_refs[1][...] = _dotT(a01, t1)
    o_refs[2][...] = _dotT(a02, t1)
    o_refs[3][...] = _dotT(a12, t1)
    o_refs[4][...] = _dotT(s00, t2)
    o_refs[5][...] = _dotT(s11, t2)
    o_refs[6][...] = _dotT(s01, t2)
    o_refs[7][...] = _dotT(s02, t2)
    o_refs[8][...] = _dotT(s12, t2)


def _run_prep(Xt, T0, T1, T2):
    N, _, H = Xt.shape
    BN = 1000
    grid = (N // BN,)
    outs = pl.pallas_call(
        _prep_body,
        grid=grid,
        in_specs=[
            pl.BlockSpec((BN, 9, H), lambda i: (i, 0, 0)),
            pl.BlockSpec((H, H), lambda i: (0, 0)),
            pl.BlockSpec((H, H), lambda i: (0, 0)),
            pl.BlockSpec((H, H), lambda i: (0, 0)),
        ],
        out_specs=[pl.BlockSpec((BN, H), lambda i: (i, 0))] * 9,
        out_shape=[jax.ShapeDtypeStruct((N, H), jnp.float32)] * 9,
    )(Xt, T0, T1, T2)
    return outs


def _make_sc_mp(N, E, H, n_batch, B):
    NT = 32
    ept = E // NT
    nh = n_batch // 2
    rpt = 632
    tail = N - 15 * rpt
    mesh = plsc.VectorSubcoreMesh(core_axis_name="c", subcore_axis_name="s")

    @functools.partial(
        pl.kernel,
        mesh=mesh,
        out_type=[jax.ShapeDtypeStruct((2, N, H), jnp.float32)] * 9,
        scratch_types=[
            pltpu.VMEM((nh, B), jnp.int32),
            pltpu.VMEM((nh, B), jnp.int32),
            pltpu.VMEM((2, B, H), jnp.float32),
            pltpu.VMEM((2, B, H), jnp.float32),
            pltpu.VMEM_SHARED((N, H), jnp.float32),
            pltpu.SemaphoreType.DMA,
            pltpu.SemaphoreType.DMA,
            pltpu.SemaphoreType.DMA,
            pltpu.SemaphoreType.DMA,
        ],
    )
    def sc_mp(src_hbm, dst_hbm, z_hbm, ea0, ea1, ea2, g0, g1, g2, g3, g4,
              g5, g6, g7, g8, o0, o1, o2, o3, o4, o5, o6, o7, o8,
              src_v, dst_v, rows_v, coef_v, acc,
              sg0, sg1, sc0, sc1):
        ea_refs = (ea0, ea1, ea2)
        g_refs = (g0, g1, g2, g3, g4, g5, g6, g7, g8)
        o_refs = (o0, o1, o2, o3, o4, o5, o6, o7, o8)
        sg = (sg0, sg1)
        sc = (sc0, sc1)
        cid = lax.axis_index("c")
        sid = lax.axis_index("s")
        wid = cid * 16 + sid

        def issue(comp, p, j, half):
            pltpu.async_copy(g_refs[comp].at[src_v.at[j]], rows_v.at[p],
                             sg[p])
            pltpu.async_copy(
                ea_refs[_GRP[comp]].at[
                    pl.ds(wid * ept + (half * nh + j) * B, B)],
                coef_v.at[p], sc[p])

        for comp in range(9):
            @pl.when(sid < 15)
            def _():
                pltpu.sync_copy(z_hbm.at[pl.ds(0, rpt)],
                                acc.at[pl.ds(sid * rpt, rpt)])

            @pl.when(sid == 15)
            def _():
                pltpu.sync_copy(z_hbm.at[pl.ds(0, tail)],
                                acc.at[pl.ds(15 * rpt, tail)])
            plsc.subcore_barrier()

            for half in range(2):
                pltpu.sync_copy(src_hbm.at[wid, pl.ds(half * nh, nh)],
                                src_v)
                pltpu.sync_copy(dst_hbm.at[wid, pl.ds(half * nh, nh)],
                                dst_v)
                issue(comp, 0, 0, half)
                issue(comp, 1, 1, half)

                def batch2(jj, _):
                    for p in range(2):
                        j = jj * 2 + p
                        rows = rows_v.at[p]
                        coef = coef_v.at[p]
                        pltpu.make_async_copy(
                            g_refs[comp].at[src_v.at[j]], rows,
                            sg[p]).wait()
                        pltpu.make_async_copy(
                            ea_refs[_GRP[comp]].at[
                                pl.ds(wid * ept + (half * nh + j) * B, B)],
                            coef, sc[p]).wait()

                        def edge(e, _):
                            for k in range(H // 16):
                                sl = pl.ds(k * 16, 16)
                                rows[e, sl] = rows[e, sl] * coef[e, sl]
                            return 0
                        lax.fori_loop(0, B, edge, 0)
                        pltpu.sync_copy(rows, acc.at[dst_v.at[j]],
                                        add=True)

                        @pl.when(j + 2 < nh)
                        def _():
                            issue(comp, p, j + 2, half)
                    return 0
                lax.fori_loop(0, nh // 2, batch2, 0)
            plsc.subcore_barrier()
            @pl.when(sid < 15)
            def _():
                pltpu.sync_copy(
                    acc.at[pl.ds(sid * rpt, rpt)],
                    o_refs[comp].at[cid, pl.ds(sid * rpt, rpt)])

            @pl.when(sid == 15)
            def _():
                pltpu.sync_copy(
                    acc.at[pl.ds(15 * rpt, tail)],
                    o_refs[comp].at[cid, pl.ds(15 * rpt, tail)])

    return sc_mp


def _final_body(x_ref, q_ref, t3_ref, t4_ref, t5_ref, *refs):
    g = [refs[k][...] for k in range(9)]
    m = [refs[9 + k][0] + refs[9 + k][1] for k in range(9)]
    x = x_ref[...]
    tn = jnp.sum(x * x, axis=1) + 1.0
    inv = 1.0 / tn
    xn = [x[:, k, :] * inv for k in range(9)]

    def full(lam, a01, a02, a12, s00, s11, s01, s02, s12):
        s22 = -(s00 + s11)
        return [[lam + s00, a01 + s01, a02 + s02],
                [s01 - a01, lam + s11, a12 + s12],
                [s02 - a02, s12 - a12, lam + s22]]

    Y = full(*g)
    M = full(*m)
    q = q_ref[...]
    qf = 1.0 + 0.1 * q
    Z = [[None] * 3 for _ in range(3)]
    for i in range(3):
        for jj in range(3):
            acc = None
            for k in range(3):
                term = M[i][k] * Y[k][jj] + Y[i][k] * M[k][jj]
                acc = term if acc is None else acc + term
            Z[i][jj] = qf * acc
    nrm = None
    for i in range(3):
        for jj in range(3):
            t = Z[i][jj] * Z[i][jj]
            nrm = t if nrm is None else nrm + t
    invn = 1.0 / (nrm + 1.0)
    lam = (Z[0][0] + Z[1][1] + Z[2][2]) * (1.0 / 3.0) * invn
    a01 = 0.5 * (Z[0][1] - Z[1][0]) * invn
    a02 = 0.5 * (Z[0][2] - Z[2][0]) * invn
    a12 = 0.5 * (Z[1][2] - Z[2][1]) * invn
    s00 = Z[0][0] * invn - lam
    s11 = Z[1][1] * invn - lam
    s01 = 0.5 * (Z[0][1] + Z[1][0]) * invn
    s02 = 0.5 * (Z[0][2] + Z[2][0]) * invn
    s12 = 0.5 * (Z[1][2] + Z[2][1]) * invn
    t3 = t3_ref[...]
    t4 = t4_ref[...]
    t5 = t5_ref[...]
    D = full(_dotT(lam, t3), _dotT(a01, t4), _dotT(a02, t4), _dotT(a12, t4),
             _dotT(s00, t5), _dotT(s11, t5), _dotT(s01, t5), _dotT(s02, t5),
             _dotT(s12, t5))
    o_ref = refs[18]
    for i in range(3):
        for jj in range(3):
            dd = None
            for k in range(3):
                t = D[i][k] * D[k][jj]
                dd = t if dd is None else dd + t
            o_ref[:, 3 * i + jj, :] = xn[3 * i + jj] + D[i][jj] + qf * dd


def _run_final(Xt, q, T3, T4, T5, g_comps, m_parts):
    N, _, H = Xt.shape
    BN = 200
    grid = (N // BN,)
    out = pl.pallas_call(
        _final_body,
        grid=grid,
        in_specs=[
            pl.BlockSpec((BN, 9, H), lambda i: (i, 0, 0)),
            pl.BlockSpec((BN, 1), lambda i: (i, 0)),
            pl.BlockSpec((H, H), lambda i: (0, 0)),
            pl.BlockSpec((H, H), lambda i: (0, 0)),
            pl.BlockSpec((H, H), lambda i: (0, 0)),
        ] + [pl.BlockSpec((BN, H), lambda i: (i, 0))] * 9
          + [pl.BlockSpec((2, BN, H), lambda i: (0, i, 0))] * 9,
        out_specs=pl.BlockSpec((BN, 9, H), lambda i: (i, 0, 0)),
        out_shape=jax.ShapeDtypeStruct((N, 9, H), jnp.float32),
    )(Xt, q.reshape(N, 1), T3, T4, T5, *g_comps, *m_parts)
    return out


def kernel(X, edge_index, edge_weight, edge_attr, q,
           W0, b0, W1, b1, W2, b2, T0, T1, T2, T3, T4, T5):
    N, H = X.shape[0], X.shape[1]
    E = edge_index.shape[1]
    NT = 32
    B = 40
    n_batch = -(-E // (NT * B))
    n_batch += (-n_batch) % 4
    Ep = NT * B * n_batch
    pad = Ep - E

    ea_in = jnp.pad(edge_attr, ((0, pad), (0, 0)))
    ew_in = jnp.pad(edge_weight, (0, pad), constant_values=2.0 * CUTOFF)
    ea = _run_mlp(ea_in, ew_in, W0, b0, W1, b1, W2, b2)

    Xt = X.reshape(N, H, 9).transpose(0, 2, 1)
    g_comps = _run_prep(Xt, T0, T1, T2)

    ei = jnp.pad(edge_index.astype(jnp.int32), ((0, 0), (0, pad)))
    dst = ei[0].reshape(NT, n_batch, B)
    src = ei[1].reshape(NT, n_batch, B)
    sc_mp = _make_sc_mp(N, Ep, H, n_batch, B)
    zrows = jnp.zeros((640, H), jnp.float32)
    m_parts = sc_mp(src, dst, zrows, *ea, *g_comps)

    out = _run_final(Xt, q, T3, T4, T5, g_comps, m_parts)
    return out.transpose(0, 2, 1).reshape(N, H, 3, 3)
